# baseline (speedup 1.0000x reference)
import functools

import jax
import jax.numpy as jnp
from jax import lax
from jax.experimental import pallas as pl
from jax.experimental.pallas import tpu as pltpu
from jax.experimental.pallas import tpu_sc as plsc

N_NODES = 50000
HID = 64
HALF = 32
N_EDGES = 800000
N_LABEL = 100000

CH = 125
GRP = 8
NCHUNK = N_EDGES // CH
NSUB = 16
NCORE = 2
CPT = NCHUNK // NSUB
GPT = CPT // GRP
ZCH = 500
NZCH = N_NODES // ZCH
SGRP = 4
SGPT = CPT // SGRP
NLCHUNK = N_LABEL // CH
NLGRP = NLCHUNK // GRP

BR = 5000
TCGRID = N_NODES // BR

_f32 = jnp.float32


def _sc_mesh():
    return plsc.VectorSubcoreMesh(core_axis_name="c", subcore_axis_name="s")


@functools.partial(
    pl.kernel,
    out_type=jax.ShapeDtypeStruct((NCORE, N_NODES), _f32),
    mesh=_sc_mesh(),
    compiler_params=pltpu.CompilerParams(use_tc_tiling_on_sc=False),
    scratch_types=[
        pltpu.VMEM_SHARED((N_NODES,), _f32),
        pltpu.VMEM((128,), _f32),
        pltpu.VMEM((1024,), _f32),
        pltpu.VMEM((GRP, CH), jnp.int32),
    ],
)
def _sc_degree(dst2, degp, acc, ones_v, zb, didx):
    c = lax.axis_index("c")
    s = lax.axis_index("s")
    o16 = jnp.ones((16,), _f32)
    z16 = jnp.zeros((16,), _f32)

    @pl.loop(0, 8)
    def _(r):
        ones_v[pl.ds(r * 16, 16)] = o16

    @pl.loop(0, 64)
    def _(r):
        zb[pl.ds(r * 16, 16)] = z16

    for k in range(N_NODES // 1000 // NSUB + 1):
        zc = s + k * NSUB

        @pl.when(zc < N_NODES // 1000)
        def _():
            pltpu.sync_copy(zb.at[pl.ds(0, 1000)], acc.at[pl.ds(zc * 1000, 1000)])

    plsc.subcore_barrier()

    @pl.loop(0, CPT // (NCORE * GRP))
    def _(g):
        m0 = c * (NCHUNK // NCORE) + s * (CPT // NCORE) + g * GRP
        pltpu.sync_copy(dst2.at[pl.ds(m0, GRP), :], didx)
        for j in range(GRP):
            pltpu.sync_copy(ones_v.at[pl.ds(0, CH)], acc.at[didx.at[j]], add=True)

    plsc.subcore_barrier()

    @pl.when(s == 0)
    def _():
        pltpu.sync_copy(acc, degp.at[c])


@functools.partial(
    pl.kernel,
    out_type=[
        jax.ShapeDtypeStruct((N_NODES, HALF), _f32),
        jax.ShapeDtypeStruct((N_NODES, HALF), _f32),
    ],
    mesh=_sc_mesh(),
    compiler_params=pltpu.CompilerParams(use_tc_tiling_on_sc=False),
    scratch_types=[
        pltpu.VMEM_SHARED((N_NODES, HALF), _f32),
        pltpu.VMEM((4, 2, 2, CH), jnp.int32),
        pltpu.VMEM((SGRP * CH, HALF), _f32),
        pltpu.SemaphoreType.DMA,
        pltpu.SemaphoreType.DMA,
        pltpu.SemaphoreType.DMA,
        pltpu.SemaphoreType.DMA,
        pltpu.SemaphoreType.DMA,
        pltpu.SemaphoreType.DMA,
    ],
)
def _sc_scatter(ya, yb, es, agga, aggb, acc, eidx, rows,
                gsem0, gsem1, isem0, isem1, ssem0, ssem1):
    c = lax.axis_index("c")
    s = lax.axis_index("s")
    z16 = jnp.zeros((16,), _f32)
    gsem = (gsem0, gsem1)
    isem = (isem0, isem1)
    ssem = (ssem0, ssem1)

    @pl.loop(0, SGRP * CH)
    def _(r):
        rows[r, 0:16] = z16
        rows[r, 16:32] = z16

    for k in range(NZCH // NSUB + 1):
        zc = s + k * NSUB

        @pl.when(zc < NZCH)
        def _():
            pltpu.sync_copy(
                rows.at[pl.ds(0, ZCH), :], acc.at[pl.ds(zc * ZCH, ZCH), :]
            )

    plsc.subcore_barrier()

    def half(y, agg):
        NP = CPT // 2

        def idx_start(g, p, d):
            m0 = s * CPT + g * 2
            pltpu.async_copy(es.at[pl.ds(m0, 2), :, :], eidx.at[d], isem[p])

        def idx_wait(g, p, d):
            m0 = s * CPT + g * 2
            pltpu.make_async_copy(
                es.at[pl.ds(m0, 2), :, :], eidx.at[d], isem[p]
            ).wait()

        def gath_start(p, d):
            for j in range(2):
                jj = 2 * p + j
                pltpu.async_copy(
                    y.at[eidx.at[d, j, 0]], rows.at[pl.ds(jj * CH, CH), :],
                    gsem[p],
                )

        def gath_wait(p, d):
            for j in range(2):
                jj = 2 * p + j
                pltpu.make_async_copy(
                    y.at[eidx.at[d, j, 0]], rows.at[pl.ds(jj * CH, CH), :],
                    gsem[p],
                ).wait()

        def scat_start(p, d):
            for j in range(2):
                pltpu.async_copy(
                    rows.at[pl.ds((2 * p + j) * CH, CH), :],
                    acc.at[eidx.at[d, j, 1]], ssem[p], add=True,
                )

        def scat_wait(p, d):
            for j in range(2):
                pltpu.make_async_copy(
                    rows.at[pl.ds((2 * p + j) * CH, CH), :],
                    acc.at[eidx.at[d, j, 1]], ssem[p],
                ).wait()

        idx_start(0, 0, 0)
        idx_wait(0, 0, 0)
        gath_start(0, 0)
        idx_start(1, 1, 1)

        @pl.loop(0, NP // 4)
        def _(i):
            for u in range(4):
                g = 4 * i + u
                p = u % 2
                gath_wait(p, u)

                @pl.when(g + 1 < NP)
                def _():
                    @pl.when(g >= 1)
                    def _():
                        scat_wait(1 - p, (u - 1) % 4)

                    idx_wait(g + 1, 1 - p, (u + 1) % 4)
                    gath_start(1 - p, (u + 1) % 4)

                scat_start(p, u)

                @pl.when(g + 2 < NP)
                def _():
                    idx_start(g + 2, p, (u + 2) % 4)

        scat_wait((NP - 2) % 2, (NP - 2) % 4)
        scat_wait((NP - 1) % 2, (NP - 1) % 4)
        plsc.subcore_barrier()
        for k in range(NZCH // NSUB + 1):
            zc = s + k * NSUB

            @pl.when(zc < NZCH)
            def _():
                pltpu.sync_copy(
                    acc.at[pl.ds(zc * ZCH, ZCH), :],
                    agg.at[pl.ds(zc * ZCH, ZCH), :],
                )

    @pl.when(c == 0)
    def _():
        half(ya, agga)

    @pl.when(c == 1)
    def _():
        half(yb, aggb)


@functools.partial(
    pl.kernel,
    out_type=jax.ShapeDtypeStruct((NLCHUNK, 128), _f32),
    mesh=_sc_mesh(),
    compiler_params=pltpu.CompilerParams(
        use_tc_tiling_on_sc=False, needs_layout_passes=False
    ),
    scratch_types=[
        pltpu.VMEM((GRP, CH), jnp.int32),
        pltpu.VMEM((GRP, CH), jnp.int32),
        pltpu.VMEM((2, 128, HID), _f32),
        pltpu.VMEM((2, 128, HID), _f32),
        pltpu.VMEM((16, 17), _f32),
        pltpu.VMEM((GRP, 128), _f32),
        pltpu.SemaphoreType.DMA,
        pltpu.SemaphoreType.DMA,
    ],
)
def _sc_decode(z, ls2, ld2, out2, lsb, ldb, S, D, P, res, gsem0, gsem1):
    c = lax.axis_index("c")
    s = lax.axis_index("s")
    w = s * NCORE + c
    gsem = (gsem0, gsem1)
    iota = lax.iota(jnp.int32, 16)

    @pl.loop(0, NLGRP // (NCORE * NSUB) + 1)
    def _(k):
        g = w + k * NCORE * NSUB

        @pl.when(g < NLGRP)
        def _():
            m0 = g * GRP
            pltpu.sync_copy(ls2.at[pl.ds(m0, GRP), :], lsb)
            pltpu.sync_copy(ld2.at[pl.ds(m0, GRP), :], ldb)

            def gath_start(j, p):
                pltpu.async_copy(z.at[lsb.at[j]], S.at[p, pl.ds(0, CH), :], gsem[p])
                pltpu.async_copy(z.at[ldb.at[j]], D.at[p, pl.ds(0, CH), :], gsem[p])

            def gath_wait(j, p):
                pltpu.make_async_copy(
                    z.at[lsb.at[j]], S.at[p, pl.ds(0, CH), :], gsem[p]
                ).wait()
                pltpu.make_async_copy(
                    z.at[ldb.at[j]], D.at[p, pl.ds(0, CH), :], gsem[p]
                ).wait()

            gath_start(0, 0)

            @pl.loop(0, GRP // 2)
            def _(jj):
                for p in (0, 1):
                    j = 2 * jj + p
                    gath_wait(j, p)

                    @pl.when(j + 1 < GRP)
                    def _():
                        gath_start(j + 1, 1 - p)

                    @pl.loop(0, 8)
                    def _(q):
                        for i in range(16):
                            r = q * 16 + i
                            pr = S[p, r, 0:16] * D[p, r, 0:16]
                            pr = pr + S[p, r, 16:32] * D[p, r, 16:32]
                            pr = pr + S[p, r, 32:48] * D[p, r, 32:48]
                            pr = pr + S[p, r, 48:64] * D[p, r, 48:64]
                            P[i, 0:16] = pr
                        acc = jnp.zeros((16,), _f32)
                        for col in range(16):
                            ci = jnp.full((16,), col, jnp.int32)
                            acc = acc + plsc.load_gather(P, [iota, ci])
                        res[j, pl.ds(q * 16, 16)] = acc

            pltpu.sync_copy(res, out2.at[pl.ds(m0, GRP), :])


def _tc_prea_body(emb, W1, xw):
    xw[...] = jnp.dot(emb[...], W1[...], preferred_element_type=_f32)


def _tc_preb_body(degp, xw, dinv, ya, yb):
    d = degp[...]
    deg = d[:, 0:1] + d[:, 1:2] + 1.0
    di = lax.rsqrt(deg)
    y = xw[...] * di
    dinv[...] = di
    ya[...] = y[:, :HALF]
    yb[...] = y[:, HALF:]


def _tc_mid_body(agga, aggb, ya, yb, dinv, b1, W2, oa, ob):
    di = dinv[...]
    h = jnp.concatenate([agga[...] + ya[...], aggb[...] + yb[...]], axis=1)
    x = jnp.maximum(h * di + b1[...], 0.0)
    y2 = jnp.dot(x, W2[...], preferred_element_type=_f32) * di
    oa[...] = y2[:, :HALF]
    ob[...] = y2[:, HALF:]


def _tc_post_body(agga, aggb, ya, yb, dinv, b2, z):
    di = dinv[...]
    h = jnp.concatenate([agga[...] + ya[...], aggb[...] + yb[...]], axis=1)
    z[...] = h * di + b2[...]


def _row_spec(width):
    return pl.BlockSpec((BR, width), lambda i: (i, 0))


def _full_spec(shape):
    return pl.BlockSpec(shape, lambda i: tuple(0 for _ in shape))


_tc_prea = pl.pallas_call(
    _tc_prea_body,
    grid=(TCGRID,),
    in_specs=[_row_spec(HID), _full_spec((HID, HID))],
    out_specs=_row_spec(HID),
    out_shape=jax.ShapeDtypeStruct((N_NODES, HID), _f32),
)

_tc_preb = pl.pallas_call(
    _tc_preb_body,
    grid=(TCGRID,),
    in_specs=[_row_spec(NCORE), _row_spec(HID)],
    out_specs=[_row_spec(1), _row_spec(HALF), _row_spec(HALF)],
    out_shape=[
        jax.ShapeDtypeStruct((N_NODES, 1), _f32),
        jax.ShapeDtypeStruct((N_NODES, HALF), _f32),
        jax.ShapeDtypeStruct((N_NODES, HALF), _f32),
    ],
)

_tc_mid = pl.pallas_call(
    _tc_mid_body,
    grid=(TCGRID,),
    in_specs=[
        _row_spec(HALF),
        _row_spec(HALF),
        _row_spec(HALF),
        _row_spec(HALF),
        _row_spec(1),
        _full_spec((1, HID)),
        _full_spec((HID, HID)),
    ],
    out_specs=[_row_spec(HALF), _row_spec(HALF)],
    out_shape=[
        jax.ShapeDtypeStruct((N_NODES, HALF), _f32),
        jax.ShapeDtypeStruct((N_NODES, HALF), _f32),
    ],
)

_tc_post = pl.pallas_call(
    _tc_post_body,
    grid=(TCGRID,),
    in_specs=[
        _row_spec(HALF),
        _row_spec(HALF),
        _row_spec(HALF),
        _row_spec(HALF),
        _row_spec(1),
        _full_spec((1, HID)),
    ],
    out_specs=_row_spec(HID),
    out_shape=jax.ShapeDtypeStruct((N_NODES, HID), _f32),
)

@jax.jit
def kernel(edge_label_index, edge_index, emb, W1, b1, W2, b2):
    es = jnp.stack(
        [edge_index[0].reshape(NCHUNK, CH), edge_index[1].reshape(NCHUNK, CH)],
        axis=1,
    )
    dst2 = edge_index[1].reshape(NCHUNK, CH)
    ls2 = edge_label_index[0].reshape(NLCHUNK, CH)
    ld2 = edge_label_index[1].reshape(NLCHUNK, CH)

    xw1 = _tc_prea(emb, W1.astype(_f32))
    degp = _sc_degree(dst2)
    dinv, y1a, y1b = _tc_preb(degp.T, xw1)
    agg1a, agg1b = _sc_scatter(y1a, y1b, es)
    y2a, y2b = _tc_mid(agg1a, agg1b, y1a, y1b, dinv,
                       b1.reshape(1, HID).astype(_f32), W2.astype(_f32))
    agg2a, agg2b = _sc_scatter(y2a, y2b, es)
    z = _tc_post(agg2a, agg2b, y2a, y2b, dinv, b2.reshape(1, HID).astype(_f32))
    out2 = _sc_decode(z, ls2, ld2)
    return out2[:, :CH].reshape(N_LABEL)


# speedup vs baseline: 1.0246x; 1.0246x over previous
import functools

import jax
import jax.numpy as jnp
from jax import lax
from jax.experimental import pallas as pl
from jax.experimental.pallas import tpu as pltpu
from jax.experimental.pallas import tpu_sc as plsc

N_NODES = 50000
HID = 64
HALF = 32
N_EDGES = 800000
N_LABEL = 100000

CH = 125
GRP = 8
NCHUNK = N_EDGES // CH
NSUB = 16
NCORE = 2
CPT = NCHUNK // NSUB
GPT = CPT // GRP
ZCH = 500
NZCH = N_NODES // ZCH
SGRP = 4
SGPT = CPT // SGRP
NLCHUNK = N_LABEL // CH
NLGRP = NLCHUNK // GRP

BR = 5000
TCGRID = N_NODES // BR

_f32 = jnp.float32


def _sc_mesh():
    return plsc.VectorSubcoreMesh(core_axis_name="c", subcore_axis_name="s")


@functools.partial(
    pl.kernel,
    out_type=jax.ShapeDtypeStruct((NCORE, N_NODES), _f32),
    mesh=_sc_mesh(),
    compiler_params=pltpu.CompilerParams(use_tc_tiling_on_sc=False),
    scratch_types=[
        pltpu.VMEM_SHARED((N_NODES,), _f32),
        pltpu.VMEM((128,), _f32),
        pltpu.VMEM((1024,), _f32),
        pltpu.VMEM((GRP, CH), jnp.int32),
    ],
)
def _sc_degree(dst2, degp, acc, ones_v, zb, didx):
    c = lax.axis_index("c")
    s = lax.axis_index("s")
    o16 = jnp.ones((16,), _f32)
    z16 = jnp.zeros((16,), _f32)

    @pl.loop(0, 8)
    def _(r):
        ones_v[pl.ds(r * 16, 16)] = o16

    @pl.loop(0, 64)
    def _(r):
        zb[pl.ds(r * 16, 16)] = z16

    for k in range(N_NODES // 1000 // NSUB + 1):
        zc = s + k * NSUB

        @pl.when(zc < N_NODES // 1000)
        def _():
            pltpu.sync_copy(zb.at[pl.ds(0, 1000)], acc.at[pl.ds(zc * 1000, 1000)])

    plsc.subcore_barrier()

    @pl.loop(0, CPT // (NCORE * GRP))
    def _(g):
        m0 = c * (NCHUNK // NCORE) + s * (CPT // NCORE) + g * GRP
        pltpu.sync_copy(dst2.at[pl.ds(m0, GRP), :], didx)
        for j in range(GRP):
            pltpu.sync_copy(ones_v.at[pl.ds(0, CH)], acc.at[didx.at[j]], add=True)

    plsc.subcore_barrier()

    @pl.when(s == 0)
    def _():
        pltpu.sync_copy(acc, degp.at[c])


@functools.partial(
    pl.kernel,
    out_type=[
        jax.ShapeDtypeStruct((N_NODES, HALF), _f32),
        jax.ShapeDtypeStruct((N_NODES, HALF), _f32),
    ],
    mesh=_sc_mesh(),
    compiler_params=pltpu.CompilerParams(use_tc_tiling_on_sc=False),
    scratch_types=[
        pltpu.VMEM_SHARED((N_NODES, HALF), _f32),
        pltpu.VMEM((4, 2, 2, CH), jnp.int32),
        pltpu.VMEM((SGRP * CH, HALF), _f32),
        pltpu.SemaphoreType.DMA,
        pltpu.SemaphoreType.DMA,
        pltpu.SemaphoreType.DMA,
        pltpu.SemaphoreType.DMA,
        pltpu.SemaphoreType.DMA,
        pltpu.SemaphoreType.DMA,
    ],
)
def _sc_scatter(ya, yb, es, agga, aggb, acc, eidx, rows,
                gsem0, gsem1, isem0, isem1, ssem0, ssem1):
    c = lax.axis_index("c")
    s = lax.axis_index("s")
    z16 = jnp.zeros((16,), _f32)
    gsem = (gsem0, gsem1)
    isem = (isem0, isem1)
    ssem = (ssem0, ssem1)

    @pl.loop(0, SGRP * CH)
    def _(r):
        rows[r, 0:16] = z16
        rows[r, 16:32] = z16

    for k in range(NZCH // NSUB + 1):
        zc = s + k * NSUB

        @pl.when(zc < NZCH)
        def _():
            pltpu.sync_copy(
                rows.at[pl.ds(0, ZCH), :], acc.at[pl.ds(zc * ZCH, ZCH), :]
            )

    plsc.subcore_barrier()

    def half(y, agg):
        NP = CPT // 2

        def idx_start(g, p, d):
            m0 = s * CPT + g * 2
            pltpu.async_copy(es.at[pl.ds(m0, 2), :, :], eidx.at[d], isem[p])

        def idx_wait(g, p, d):
            m0 = s * CPT + g * 2
            pltpu.make_async_copy(
                es.at[pl.ds(m0, 2), :, :], eidx.at[d], isem[p]
            ).wait()

        def gath_start(p, d):
            for j in range(2):
                jj = 2 * p + j
                pltpu.async_copy(
                    y.at[eidx.at[d, j, 0]], rows.at[pl.ds(jj * CH, CH), :],
                    gsem[p],
                )

        def gath_wait(p, d):
            for j in range(2):
                jj = 2 * p + j
                pltpu.make_async_copy(
                    y.at[eidx.at[d, j, 0]], rows.at[pl.ds(jj * CH, CH), :],
                    gsem[p],
                ).wait()

        def scat_start(p, d):
            for j in range(2):
                pltpu.async_copy(
                    rows.at[pl.ds((2 * p + j) * CH, CH), :],
                    acc.at[eidx.at[d, j, 1]], ssem[p], add=True,
                )

        def scat_wait(p, d):
            for j in range(2):
                pltpu.make_async_copy(
                    rows.at[pl.ds((2 * p + j) * CH, CH), :],
                    acc.at[eidx.at[d, j, 1]], ssem[p],
                ).wait()

        idx_start(0, 0, 0)
        idx_wait(0, 0, 0)
        gath_start(0, 0)
        idx_start(1, 1, 1)

        @pl.loop(0, NP // 4)
        def _(i):
            for u in range(4):
                g = 4 * i + u
                p = u % 2
                gath_wait(p, u)

                @pl.when(g + 1 < NP)
                def _():
                    @pl.when(g >= 1)
                    def _():
                        scat_wait(1 - p, (u - 1) % 4)

                    idx_wait(g + 1, 1 - p, (u + 1) % 4)
                    gath_start(1 - p, (u + 1) % 4)

                scat_start(p, u)

                @pl.when(g + 2 < NP)
                def _():
                    idx_start(g + 2, p, (u + 2) % 4)

        scat_wait((NP - 2) % 2, (NP - 2) % 4)
        scat_wait((NP - 1) % 2, (NP - 1) % 4)
        plsc.subcore_barrier()
        for k in range(NZCH // NSUB + 1):
            zc = s + k * NSUB

            @pl.when(zc < NZCH)
            def _():
                pltpu.sync_copy(
                    acc.at[pl.ds(zc * ZCH, ZCH), :],
                    agg.at[pl.ds(zc * ZCH, ZCH), :],
                )

    @pl.when(c == 0)
    def _():
        half(ya, agga)

    @pl.when(c == 1)
    def _():
        half(yb, aggb)


@functools.partial(
    pl.kernel,
    out_type=jax.ShapeDtypeStruct((NLCHUNK, 128), _f32),
    mesh=_sc_mesh(),
    compiler_params=pltpu.CompilerParams(
        use_tc_tiling_on_sc=False, needs_layout_passes=False
    ),
    scratch_types=[
        pltpu.VMEM((GRP, CH), jnp.int32),
        pltpu.VMEM((GRP, CH), jnp.int32),
        pltpu.VMEM((2, 128, HID), _f32),
        pltpu.VMEM((2, 128, HID), _f32),
        pltpu.VMEM((16, 17), _f32),
        pltpu.VMEM((GRP, 128), _f32),
        pltpu.SemaphoreType.DMA,
        pltpu.SemaphoreType.DMA,
    ],
)
def _sc_decode(z, ls2, ld2, out2, lsb, ldb, S, D, P, res, gsem0, gsem1):
    c = lax.axis_index("c")
    s = lax.axis_index("s")
    w = s * NCORE + c
    gsem = (gsem0, gsem1)
    iota = lax.iota(jnp.int32, 16)

    @pl.loop(0, NLGRP // (NCORE * NSUB) + 1)
    def _(k):
        g = w + k * NCORE * NSUB

        @pl.when(g < NLGRP)
        def _():
            m0 = g * GRP
            pltpu.sync_copy(ls2.at[pl.ds(m0, GRP), :], lsb)
            pltpu.sync_copy(ld2.at[pl.ds(m0, GRP), :], ldb)

            def gath_start(j, p):
                pltpu.async_copy(z.at[lsb.at[j]], S.at[p, pl.ds(0, CH), :], gsem[p])
                pltpu.async_copy(z.at[ldb.at[j]], D.at[p, pl.ds(0, CH), :], gsem[p])

            def gath_wait(j, p):
                pltpu.make_async_copy(
                    z.at[lsb.at[j]], S.at[p, pl.ds(0, CH), :], gsem[p]
                ).wait()
                pltpu.make_async_copy(
                    z.at[ldb.at[j]], D.at[p, pl.ds(0, CH), :], gsem[p]
                ).wait()

            gath_start(0, 0)

            @pl.loop(0, GRP // 2)
            def _(jj):
                for p in (0, 1):
                    j = 2 * jj + p
                    gath_wait(j, p)

                    @pl.when(j + 1 < GRP)
                    def _():
                        gath_start(j + 1, 1 - p)

                    @pl.loop(0, 8)
                    def _(q):
                        for i in range(16):
                            r = q * 16 + i
                            pr = S[p, r, 0:16] * D[p, r, 0:16]
                            pr = pr + S[p, r, 16:32] * D[p, r, 16:32]
                            pr = pr + S[p, r, 32:48] * D[p, r, 32:48]
                            pr = pr + S[p, r, 48:64] * D[p, r, 48:64]
                            P[i, 0:16] = pr
                        acc = jnp.zeros((16,), _f32)
                        for col in range(16):
                            ci = jnp.full((16,), col, jnp.int32)
                            acc = acc + plsc.load_gather(P, [iota, ci])
                        res[j, pl.ds(q * 16, 16)] = acc

            pltpu.sync_copy(res, out2.at[pl.ds(m0, GRP), :])


def _tc_pre_body(degp, emb, W1, dinv, ya, yb):
    d = degp[...]
    deg = d[:, 0:1] + d[:, 1:2] + 1.0
    di = lax.rsqrt(deg)
    y = jnp.dot(emb[...], W1[...], preferred_element_type=_f32) * di
    dinv[...] = di
    ya[...] = y[:, :HALF]
    yb[...] = y[:, HALF:]


def _tc_mid_body(agga, aggb, ya, yb, dinv, b1, W2, oa, ob):
    di = dinv[...]
    h = jnp.concatenate([agga[...] + ya[...], aggb[...] + yb[...]], axis=1)
    x = jnp.maximum(h * di + b1[...], 0.0)
    y2 = jnp.dot(x, W2[...], preferred_element_type=_f32) * di
    oa[...] = y2[:, :HALF]
    ob[...] = y2[:, HALF:]


def _tc_post_body(agga, aggb, ya, yb, dinv, b2, z):
    di = dinv[...]
    h = jnp.concatenate([agga[...] + ya[...], aggb[...] + yb[...]], axis=1)
    z[...] = h * di + b2[...]


def _row_spec(width):
    return pl.BlockSpec((BR, width), lambda i: (i, 0))


def _full_spec(shape):
    return pl.BlockSpec(shape, lambda i: tuple(0 for _ in shape))


_tc_pre = pl.pallas_call(
    _tc_pre_body,
    grid=(TCGRID,),
    in_specs=[
        _row_spec(NCORE),
        _row_spec(HID),
        _full_spec((HID, HID)),
    ],
    out_specs=[_row_spec(1), _row_spec(HALF), _row_spec(HALF)],
    out_shape=[
        jax.ShapeDtypeStruct((N_NODES, 1), _f32),
        jax.ShapeDtypeStruct((N_NODES, HALF), _f32),
        jax.ShapeDtypeStruct((N_NODES, HALF), _f32),
    ],
)

_tc_mid = pl.pallas_call(
    _tc_mid_body,
    grid=(TCGRID,),
    in_specs=[
        _row_spec(HALF),
        _row_spec(HALF),
        _row_spec(HALF),
        _row_spec(HALF),
        _row_spec(1),
        _full_spec((1, HID)),
        _full_spec((HID, HID)),
    ],
    out_specs=[_row_spec(HALF), _row_spec(HALF)],
    out_shape=[
        jax.ShapeDtypeStruct((N_NODES, HALF), _f32),
        jax.ShapeDtypeStruct((N_NODES, HALF), _f32),
    ],
)

_tc_post = pl.pallas_call(
    _tc_post_body,
    grid=(TCGRID,),
    in_specs=[
        _row_spec(HALF),
        _row_spec(HALF),
        _row_spec(HALF),
        _row_spec(HALF),
        _row_spec(1),
        _full_spec((1, HID)),
    ],
    out_specs=_row_spec(HID),
    out_shape=jax.ShapeDtypeStruct((N_NODES, HID), _f32),
)

@jax.jit
def kernel(edge_label_index, edge_index, emb, W1, b1, W2, b2):
    es = jnp.stack(
        [edge_index[0].reshape(NCHUNK, CH), edge_index[1].reshape(NCHUNK, CH)],
        axis=1,
    )
    dst2 = edge_index[1].reshape(NCHUNK, CH)
    ls2 = edge_label_index[0].reshape(NLCHUNK, CH)
    ld2 = edge_label_index[1].reshape(NLCHUNK, CH)

    degp = _sc_degree(dst2)
    dinv, y1a, y1b = _tc_pre(degp.T, emb, W1.astype(_f32))
    agg1a, agg1b = _sc_scatter(y1a, y1b, es)
    y2a, y2b = _tc_mid(agg1a, agg1b, y1a, y1b, dinv,
                       b1.reshape(1, HID).astype(_f32), W2.astype(_f32))
    agg2a, agg2b = _sc_scatter(y2a, y2b, es)
    z = _tc_post(agg2a, agg2b, y2a, y2b, dinv, b2.reshape(1, HID).astype(_f32))
    out2 = _sc_decode(z, ls2, ld2)
    return out2[:, :CH].reshape(N_LABEL)


# speedup vs baseline: 1.1101x; 1.0835x over previous
import functools

import jax
import jax.numpy as jnp
from jax import lax
from jax.experimental import pallas as pl
from jax.experimental.pallas import tpu as pltpu
from jax.experimental.pallas import tpu_sc as plsc

N_NODES = 50000
HID = 64
HALF = 32
N_EDGES = 800000
N_LABEL = 100000

CH = 125
GRP = 8
NCHUNK = N_EDGES // CH
NSUB = 16
NCORE = 2
CPT = NCHUNK // NSUB
GPT = CPT // GRP
ZCH = 500
NZCH = N_NODES // ZCH
SGRP = 4
SGPT = CPT // SGRP
NLCHUNK = N_LABEL // CH
NLGRP = NLCHUNK // GRP

BR = 5000
TCGRID = N_NODES // BR

_f32 = jnp.float32


def _sc_mesh():
    return plsc.VectorSubcoreMesh(core_axis_name="c", subcore_axis_name="s")


@functools.partial(
    pl.kernel,
    out_type=jax.ShapeDtypeStruct((NCORE, N_NODES), _f32),
    mesh=_sc_mesh(),
    compiler_params=pltpu.CompilerParams(use_tc_tiling_on_sc=False),
    scratch_types=[
        pltpu.VMEM_SHARED((N_NODES,), _f32),
        pltpu.VMEM((128,), _f32),
        pltpu.VMEM((1024,), _f32),
        pltpu.VMEM((GRP, CH), jnp.int32),
    ],
)
def _sc_degree(dst2, degp, acc, ones_v, zb, didx):
    c = lax.axis_index("c")
    s = lax.axis_index("s")
    o16 = jnp.ones((16,), _f32)
    z16 = jnp.zeros((16,), _f32)

    @pl.loop(0, 8)
    def _(r):
        ones_v[pl.ds(r * 16, 16)] = o16

    @pl.loop(0, 64)
    def _(r):
        zb[pl.ds(r * 16, 16)] = z16

    for k in range(N_NODES // 1000 // NSUB + 1):
        zc = s + k * NSUB

        @pl.when(zc < N_NODES // 1000)
        def _():
            pltpu.sync_copy(zb.at[pl.ds(0, 1000)], acc.at[pl.ds(zc * 1000, 1000)])

    plsc.subcore_barrier()

    @pl.loop(0, CPT // (NCORE * GRP))
    def _(g):
        m0 = c * (NCHUNK // NCORE) + s * (CPT // NCORE) + g * GRP
        pltpu.sync_copy(dst2.at[pl.ds(m0, GRP), :], didx)
        for j in range(GRP):
            pltpu.sync_copy(ones_v.at[pl.ds(0, CH)], acc.at[didx.at[j]], add=True)

    plsc.subcore_barrier()

    @pl.when(s == 0)
    def _():
        pltpu.sync_copy(acc, degp.at[c])


@functools.partial(
    pl.kernel,
    out_type=[
        jax.ShapeDtypeStruct((N_NODES, HALF), _f32),
        jax.ShapeDtypeStruct((N_NODES, HALF), _f32),
    ],
    mesh=_sc_mesh(),
    compiler_params=pltpu.CompilerParams(use_tc_tiling_on_sc=False),
    scratch_types=[
        pltpu.VMEM_SHARED((N_NODES, HALF), _f32),
        pltpu.VMEM((4, 3, 2, CH), jnp.int32),
        pltpu.VMEM((6 * CH, HALF), _f32),
        pltpu.SemaphoreType.DMA,
        pltpu.SemaphoreType.DMA,
        pltpu.SemaphoreType.DMA,
        pltpu.SemaphoreType.DMA,
        pltpu.SemaphoreType.DMA,
        pltpu.SemaphoreType.DMA,
    ],
)
def _sc_scatter(ya, yb, es, agga, aggb, acc, eidx, rows,
                gsem0, gsem1, isem0, isem1, ssem0, ssem1):
    c = lax.axis_index("c")
    s = lax.axis_index("s")
    z16 = jnp.zeros((16,), _f32)
    gsem = (gsem0, gsem1)
    isem = (isem0, isem1)
    ssem = (ssem0, ssem1)

    @pl.loop(0, 6 * CH)
    def _(r):
        rows[r, 0:16] = z16
        rows[r, 16:32] = z16

    for k in range(NZCH // NSUB + 1):
        zc = s + k * NSUB

        @pl.when(zc < NZCH)
        def _():
            pltpu.sync_copy(
                rows.at[pl.ds(0, ZCH), :], acc.at[pl.ds(zc * ZCH, ZCH), :]
            )

    plsc.subcore_barrier()

    def half(y, agg):
        NP = 132
        SC_ = 3

        def idx_start(g, p, d):
            m0 = s * CPT + g * SC_
            pltpu.async_copy(es.at[pl.ds(m0, SC_), :, :], eidx.at[d], isem[p])

        def idx_wait(g, p, d):
            m0 = s * CPT + g * SC_
            pltpu.make_async_copy(
                es.at[pl.ds(m0, SC_), :, :], eidx.at[d], isem[p]
            ).wait()

        def gath_start(p, d):
            for j in range(SC_):
                jj = SC_ * p + j
                pltpu.async_copy(
                    y.at[eidx.at[d, j, 0]], rows.at[pl.ds(jj * CH, CH), :],
                    gsem[p],
                )

        def gath_wait(p, d):
            for j in range(SC_):
                jj = SC_ * p + j
                pltpu.make_async_copy(
                    y.at[eidx.at[d, j, 0]], rows.at[pl.ds(jj * CH, CH), :],
                    gsem[p],
                ).wait()

        def scat_start(p, d):
            for j in range(SC_):
                pltpu.async_copy(
                    rows.at[pl.ds((SC_ * p + j) * CH, CH), :],
                    acc.at[eidx.at[d, j, 1]], ssem[p], add=True,
                )

        def scat_wait(p, d):
            for j in range(SC_):
                pltpu.make_async_copy(
                    rows.at[pl.ds((SC_ * p + j) * CH, CH), :],
                    acc.at[eidx.at[d, j, 1]], ssem[p],
                ).wait()

        idx_start(0, 0, 0)
        idx_wait(0, 0, 0)
        gath_start(0, 0)
        idx_start(1, 1, 1)

        @pl.loop(0, NP // 4)
        def _(i):
            for u in range(4):
                g = 4 * i + u
                p = u % 2
                gath_wait(p, u)

                @pl.when(g + 1 < NP)
                def _():
                    @pl.when(g >= 1)
                    def _():
                        scat_wait(1 - p, (u - 1) % 4)

                    idx_wait(g + 1, 1 - p, (u + 1) % 4)
                    gath_start(1 - p, (u + 1) % 4)

                scat_start(p, u)

                @pl.when(g + 2 < NP)
                def _():
                    idx_start(g + 2, p, (u + 2) % 4)

        scat_wait((NP - 2) % 2, (NP - 2) % 4)
        scat_wait((NP - 1) % 2, (NP - 1) % 4)

        m0t = s * CPT + NP * SC_
        pltpu.sync_copy(es.at[pl.ds(m0t, SC_), :, :], eidx.at[0])
        pltpu.sync_copy(es.at[pl.ds(m0t + SC_, 1), :, :], eidx.at[1, pl.ds(0, 1)])
        tcps = [
            pltpu.async_copy(
                y.at[eidx.at[0, j, 0]], rows.at[pl.ds(j * CH, CH), :], gsem[0]
            )
            for j in range(SC_)
        ]
        tcps.append(
            pltpu.async_copy(
                y.at[eidx.at[1, 0, 0]], rows.at[pl.ds(SC_ * CH, CH), :], gsem[0]
            )
        )
        for cp_ in tcps:
            cp_.wait()
        for j in range(SC_):
            pltpu.sync_copy(
                rows.at[pl.ds(j * CH, CH), :], acc.at[eidx.at[0, j, 1]], add=True
            )
        pltpu.sync_copy(
            rows.at[pl.ds(SC_ * CH, CH), :], acc.at[eidx.at[1, 0, 1]], add=True
        )
        plsc.subcore_barrier()
        for k in range(NZCH // NSUB + 1):
            zc = s + k * NSUB

            @pl.when(zc < NZCH)
            def _():
                pltpu.sync_copy(
                    acc.at[pl.ds(zc * ZCH, ZCH), :],
                    agg.at[pl.ds(zc * ZCH, ZCH), :],
                )

    @pl.when(c == 0)
    def _():
        half(ya, agga)

    @pl.when(c == 1)
    def _():
        half(yb, aggb)


@functools.partial(
    pl.kernel,
    out_type=jax.ShapeDtypeStruct((NLCHUNK, 128), _f32),
    mesh=_sc_mesh(),
    compiler_params=pltpu.CompilerParams(
        use_tc_tiling_on_sc=False, needs_layout_passes=False
    ),
    scratch_types=[
        pltpu.VMEM((GRP, CH), jnp.int32),
        pltpu.VMEM((GRP, CH), jnp.int32),
        pltpu.VMEM((2, 128, HID), _f32),
        pltpu.VMEM((2, 128, HID), _f32),
        pltpu.VMEM((16, 17), _f32),
        pltpu.VMEM((GRP, 128), _f32),
        pltpu.SemaphoreType.DMA,
        pltpu.SemaphoreType.DMA,
    ],
)
def _sc_decode(z, ls2, ld2, out2, lsb, ldb, S, D, P, res, gsem0, gsem1):
    c = lax.axis_index("c")
    s = lax.axis_index("s")
    w = s * NCORE + c
    gsem = (gsem0, gsem1)
    iota = lax.iota(jnp.int32, 16)

    @pl.loop(0, NLGRP // (NCORE * NSUB) + 1)
    def _(k):
        g = w + k * NCORE * NSUB

        @pl.when(g < NLGRP)
        def _():
            m0 = g * GRP
            pltpu.sync_copy(ls2.at[pl.ds(m0, GRP), :], lsb)
            pltpu.sync_copy(ld2.at[pl.ds(m0, GRP), :], ldb)

            def gath_start(j, p):
                pltpu.async_copy(z.at[lsb.at[j]], S.at[p, pl.ds(0, CH), :], gsem[p])
                pltpu.async_copy(z.at[ldb.at[j]], D.at[p, pl.ds(0, CH), :], gsem[p])

            def gath_wait(j, p):
                pltpu.make_async_copy(
                    z.at[lsb.at[j]], S.at[p, pl.ds(0, CH), :], gsem[p]
                ).wait()
                pltpu.make_async_copy(
                    z.at[ldb.at[j]], D.at[p, pl.ds(0, CH), :], gsem[p]
                ).wait()

            gath_start(0, 0)

            @pl.loop(0, GRP // 2)
            def _(jj):
                for p in (0, 1):
                    j = 2 * jj + p
                    gath_wait(j, p)

                    @pl.when(j + 1 < GRP)
                    def _():
                        gath_start(j + 1, 1 - p)

                    @pl.loop(0, 8)
                    def _(q):
                        for i in range(16):
                            r = q * 16 + i
                            pr = S[p, r, 0:16] * D[p, r, 0:16]
                            pr = pr + S[p, r, 16:32] * D[p, r, 16:32]
                            pr = pr + S[p, r, 32:48] * D[p, r, 32:48]
                            pr = pr + S[p, r, 48:64] * D[p, r, 48:64]
                            P[i, 0:16] = pr
                        acc = jnp.zeros((16,), _f32)
                        for col in range(16):
                            ci = jnp.full((16,), col, jnp.int32)
                            acc = acc + plsc.load_gather(P, [iota, ci])
                        res[j, pl.ds(q * 16, 16)] = acc

            pltpu.sync_copy(res, out2.at[pl.ds(m0, GRP), :])


def _tc_pre_body(degp, emb, W1, dinv, ya, yb):
    d = degp[...]
    deg = d[:, 0:1] + d[:, 1:2] + 1.0
    di = lax.rsqrt(deg)
    y = jnp.dot(emb[...], W1[...], preferred_element_type=_f32) * di
    dinv[...] = di
    ya[...] = y[:, :HALF]
    yb[...] = y[:, HALF:]


def _tc_mid_body(agga, aggb, ya, yb, dinv, b1, W2, oa, ob):
    di = dinv[...]
    h = jnp.concatenate([agga[...] + ya[...], aggb[...] + yb[...]], axis=1)
    x = jnp.maximum(h * di + b1[...], 0.0)
    y2 = jnp.dot(x, W2[...], preferred_element_type=_f32) * di
    oa[...] = y2[:, :HALF]
    ob[...] = y2[:, HALF:]


def _tc_post_body(agga, aggb, ya, yb, dinv, b2, z):
    di = dinv[...]
    h = jnp.concatenate([agga[...] + ya[...], aggb[...] + yb[...]], axis=1)
    z[...] = h * di + b2[...]


def _row_spec(width):
    return pl.BlockSpec((BR, width), lambda i: (i, 0))


def _full_spec(shape):
    return pl.BlockSpec(shape, lambda i: tuple(0 for _ in shape))


_tc_pre = pl.pallas_call(
    _tc_pre_body,
    grid=(TCGRID,),
    in_specs=[
        _row_spec(NCORE),
        _row_spec(HID),
        _full_spec((HID, HID)),
    ],
    out_specs=[_row_spec(1), _row_spec(HALF), _row_spec(HALF)],
    out_shape=[
        jax.ShapeDtypeStruct((N_NODES, 1), _f32),
        jax.ShapeDtypeStruct((N_NODES, HALF), _f32),
        jax.ShapeDtypeStruct((N_NODES, HALF), _f32),
    ],
)

_tc_mid = pl.pallas_call(
    _tc_mid_body,
    grid=(TCGRID,),
    in_specs=[
        _row_spec(HALF),
        _row_spec(HALF),
        _row_spec(HALF),
        _row_spec(HALF),
        _row_spec(1),
        _full_spec((1, HID)),
        _full_spec((HID, HID)),
    ],
    out_specs=[_row_spec(HALF), _row_spec(HALF)],
    out_shape=[
        jax.ShapeDtypeStruct((N_NODES, HALF), _f32),
        jax.ShapeDtypeStruct((N_NODES, HALF), _f32),
    ],
)

_tc_post = pl.pallas_call(
    _tc_post_body,
    grid=(TCGRID,),
    in_specs=[
        _row_spec(HALF),
        _row_spec(HALF),
        _row_spec(HALF),
        _row_spec(HALF),
        _row_spec(1),
        _full_spec((1, HID)),
    ],
    out_specs=_row_spec(HID),
    out_shape=jax.ShapeDtypeStruct((N_NODES, HID), _f32),
)

@jax.jit
def kernel(edge_label_index, edge_index, emb, W1, b1, W2, b2):
    es = jnp.stack(
        [edge_index[0].reshape(NCHUNK, CH), edge_index[1].reshape(NCHUNK, CH)],
        axis=1,
    )
    dst2 = edge_index[1].reshape(NCHUNK, CH)
    ls2 = edge_label_index[0].reshape(NLCHUNK, CH)
    ld2 = edge_label_index[1].reshape(NLCHUNK, CH)

    degp = _sc_degree(dst2)
    dinv, y1a, y1b = _tc_pre(degp.T, emb, W1.astype(_f32))
    agg1a, agg1b = _sc_scatter(y1a, y1b, es)
    y2a, y2b = _tc_mid(agg1a, agg1b, y1a, y1b, dinv,
                       b1.reshape(1, HID).astype(_f32), W2.astype(_f32))
    agg2a, agg2b = _sc_scatter(y2a, y2b, es)
    z = _tc_post(agg2a, agg2b, y2a, y2b, dinv, b2.reshape(1, HID).astype(_f32))
    out2 = _sc_decode(z, ls2, ld2)
    return out2[:, :CH].reshape(N_LABEL)
